# baseline (speedup 1.0000x reference)
import functools

import jax
import jax.numpy as jnp
from jax import lax
from jax.experimental import pallas as pl
from jax.experimental.pallas import tpu as pltpu
from jax.experimental.pallas import tpu_sc as plsc

NC = 2
NS = 16
NW = NC * NS
LANES = 16
BT = 128
SPS = 2


def _build(batch, seq_len, d_model):
    steps = seq_len // SPS
    kd = d_model // LANES
    dt8 = d_model // 8
    mesh = plsc.VectorSubcoreMesh(core_axis_name="c", subcore_axis_name="s")

    @functools.partial(
        pl.kernel,
        out_type=jax.ShapeDtypeStruct(
            (seq_len * dt8 * (batch // BT) * 8 * BT,), jnp.float32),
        mesh=mesh,
        compiler_params=pltpu.CompilerParams(
            use_tc_tiling_on_sc=False, needs_layout_passes=False),
        scratch_types=[
            pltpu.VMEM((seq_len, BT), jnp.int32),
            pltpu.VMEM((SPS * BT, d_model), jnp.float32),
            pltpu.VMEM((SPS * BT, d_model), jnp.float32),
            pltpu.VMEM((SPS * dt8 * 8 * BT,), jnp.float32),
            pltpu.VMEM((SPS * dt8 * 8 * BT,), jnp.float32),
            pltpu.VMEM((seq_len, d_model), jnp.float32),
            pltpu.SemaphoreType.DMA,
            pltpu.SemaphoreType.DMA,
            pltpu.SemaphoreType.DMA,
            pltpu.SemaphoreType.DMA,
        ],
    )
    def emb(tok_hbm, table_hbm, pos_hbm, out_hbm,
            idx_v, rows0, rows1, tb0, tb1, pos_v,
            g0, g1, o0, o1):
        rows = (rows0, rows1)
        tb = (tb0, tb1)
        gsem = (g0, g1)
        osem = (o0, o1)
        wid = lax.axis_index("s") * NC + lax.axis_index("c")
        pltpu.sync_copy(pos_hbm, pos_v)
        pltpu.sync_copy(tok_hbm.at[wid], idx_v)

        def fire_gathers(g, b):
            for si in range(SPS):
                pltpu.async_copy(
                    table_hbm.at[idx_v.at[g * SPS + si]],
                    rows[b].at[pl.ds(si * BT, BT)],
                    gsem[b],
                )

        def drain_g(b):
            pltpu.make_async_copy(
                table_hbm.at[pl.ds(0, SPS * BT)], rows[b], gsem[b]).wait()

        def drain_o(b):
            pltpu.make_async_copy(
                out_hbm.at[pl.ds(0, SPS * dt8 * 8 * BT)], tb[b],
                osem[b]).wait()

        iota = lax.iota(jnp.int32, LANES)
        whi = (iota >> 3) * (8 * BT) + (iota & 7) * BT
        dvec = [iota + k * LANES for k in range(kd)]

        def transpose_add(b, g):
            for si in range(SPS):
                s = g * SPS + si
                pv = [pos_v[s, pl.ds(k * LANES, LANES)] for k in range(kd)]

                @plsc.parallel_loop(0, LANES, 1, unroll=4)
                def body(t):
                    m = (iota + t) & 15
                    wflat = whi + m
                    for blk in range(BT // LANES):
                        rowvec = m + (si * BT + blk * LANES)
                        for k in range(kd):
                            val = plsc.load_gather(
                                rows[b], [rowvec, dvec[k]]) + pv[k]
                            plsc.store_scatter(
                                tb[b],
                                [wflat + (si * dt8 * 8 * BT
                                          + k * 2 * 8 * BT + blk * LANES)],
                                val)

        def half(b, nb, g, first):
            @pl.when(g < steps - 1)
            def _():
                fire_gathers(g + 1, nb)
            drain_g(b)

            @pl.when(jnp.logical_not(first))
            def _():
                drain_o(b)
            transpose_add(b, g)
            for si in range(SPS):
                for dt in range(dt8):
                    pltpu.async_copy(
                        tb[b].at[pl.ds((si * dt8 + dt) * 8 * BT, 8 * BT)],
                        out_hbm.at[pl.ds(
                            (((g * SPS + si) * dt8 + dt) * (batch // BT)
                             + wid) * 8 * BT, 8 * BT)],
                        osem[b])

        fire_gathers(0, 0)

        def pair(p, _):
            half(0, 1, p * 2, p == 0)
            half(1, 0, p * 2 + 1, p == 0)
            return 0

        lax.fori_loop(0, steps // 2, pair, 0)
        drain_o(0)
        drain_o(1)

    return emb


def kernel(tokens, token_table, position_table):
    b, s = tokens.shape
    _, d_model = token_table.shape
    emb = _build(b, s, d_model)
    tok_w = tokens.reshape(NW, BT, s).transpose(0, 2, 1).astype(jnp.int32)
    tmp = emb(tok_w, token_table, position_table)
    tmp = tmp.reshape(s, d_model // 8, b // BT, 8, BT)
    return tmp.transpose(2, 4, 0, 1, 3).reshape(b, s, d_model)


# speedup vs baseline: 1.0500x; 1.0500x over previous
import functools

import jax
import jax.numpy as jnp
from jax import lax
from jax.experimental import pallas as pl
from jax.experimental.pallas import tpu as pltpu
from jax.experimental.pallas import tpu_sc as plsc

NC = 2
NS = 16
NW = NC * NS
LANES = 16
BT = 128
SPS = 2


def _build(batch, seq_len, d_model):
    steps = seq_len // SPS
    kd = d_model // LANES
    dt8 = d_model // 8
    mesh = plsc.VectorSubcoreMesh(core_axis_name="c", subcore_axis_name="s")

    @functools.partial(
        pl.kernel,
        out_type=jax.ShapeDtypeStruct(
            (seq_len * dt8 * (batch // BT) * 8 * BT,), jnp.float32),
        mesh=mesh,
        compiler_params=pltpu.CompilerParams(
            use_tc_tiling_on_sc=False, needs_layout_passes=False),
        scratch_types=[
            pltpu.VMEM((seq_len, BT), jnp.int32),
            pltpu.VMEM((SPS * BT, d_model), jnp.float32),
            pltpu.VMEM((SPS * BT, d_model), jnp.float32),
            pltpu.VMEM((SPS * dt8 * 8 * BT,), jnp.float32),
            pltpu.VMEM((SPS * dt8 * 8 * BT,), jnp.float32),
            pltpu.VMEM((seq_len, d_model), jnp.float32),
            pltpu.SemaphoreType.DMA,
            pltpu.SemaphoreType.DMA,
            pltpu.SemaphoreType.DMA,
            pltpu.SemaphoreType.DMA,
        ],
    )
    def emb(tok_hbm, table_hbm, pos_hbm, out_hbm,
            idx_v, rows0, rows1, tb0, tb1, pos_v,
            g0, g1, o0, o1):
        rows = (rows0, rows1)
        tb = (tb0, tb1)
        gsem = (g0, g1)
        osem = (o0, o1)
        wid = lax.axis_index("s") * NC + lax.axis_index("c")
        pltpu.sync_copy(pos_hbm, pos_v)
        pltpu.sync_copy(tok_hbm.at[wid], idx_v)

        def fire_gathers(g, b):
            for si in range(SPS):
                pltpu.async_copy(
                    table_hbm.at[idx_v.at[g * SPS + si]],
                    rows[b].at[pl.ds(si * BT, BT)],
                    gsem[b],
                )

        def drain_g(b):
            pltpu.make_async_copy(
                table_hbm.at[pl.ds(0, SPS * BT)], rows[b], gsem[b]).wait()

        def drain_o(b):
            pltpu.make_async_copy(
                out_hbm.at[pl.ds(0, SPS * dt8 * 8 * BT)], tb[b],
                osem[b]).wait()

        iota = lax.iota(jnp.int32, LANES)
        whi = (iota >> 3) * (8 * BT) + (iota & 7) * BT
        dvec = [iota + k * LANES for k in range(kd)]

        def transpose_add(b, g):
            for si in range(SPS):
                s = g * SPS + si
                pv = [pos_v[s, pl.ds(k * LANES, LANES)] for k in range(kd)]

                @plsc.parallel_loop(0, LANES, 1, unroll=2)
                def body(t):
                    m = (iota + t) & 15
                    wflat = whi + m
                    for blk in range(BT // LANES):
                        rowvec = m + (si * BT + blk * LANES)
                        for k in range(kd):
                            val = plsc.load_gather(
                                rows[b], [rowvec, dvec[k]]) + pv[k]
                            plsc.store_scatter(
                                tb[b],
                                [wflat + (si * dt8 * 8 * BT
                                          + k * 2 * 8 * BT + blk * LANES)],
                                val)

        def half(b, nb, g, first):
            @pl.when(g < steps - 1)
            def _():
                fire_gathers(g + 1, nb)
            drain_g(b)

            @pl.when(jnp.logical_not(first))
            def _():
                drain_o(b)
            transpose_add(b, g)
            for si in range(SPS):
                for dt in range(dt8):
                    pltpu.async_copy(
                        tb[b].at[pl.ds((si * dt8 + dt) * 8 * BT, 8 * BT)],
                        out_hbm.at[pl.ds(
                            (((g * SPS + si) * dt8 + dt) * (batch // BT)
                             + wid) * 8 * BT, 8 * BT)],
                        osem[b])

        fire_gathers(0, 0)

        def pair(p, _):
            half(0, 1, p * 2, p == 0)
            half(1, 0, p * 2 + 1, p == 0)
            return 0

        lax.fori_loop(0, steps // 2, pair, 0)
        drain_o(0)
        drain_o(1)

    return emb


def kernel(tokens, token_table, position_table):
    b, s = tokens.shape
    _, d_model = token_table.shape
    emb = _build(b, s, d_model)
    tok_w = tokens.reshape(NW, BT, s).transpose(0, 2, 1).astype(jnp.int32)
    tmp = emb(tok_w, token_table, position_table)
    tmp = tmp.reshape(s, d_model // 8, b // BT, 8, BT)
    return tmp.transpose(2, 4, 0, 1, 3).reshape(b, s, d_model)
